# baseline (speedup 1.0000x reference)
import functools

import jax
import jax.numpy as jnp
from jax import lax
from jax.experimental import pallas as pl
from jax.experimental.pallas import tpu as pltpu
from jax.experimental.pallas import tpu_sc as plsc

B = 4096
S = 200
D = 64
C = 2
L = 16
SP = 208
NC = 2
NS = 16
NW = NC * NS
RW = B // NW
HALF = SP // 2
NCH = SP // L

_mesh = plsc.VectorSubcoreMesh(
    core_axis_name="c", subcore_axis_name="s", num_cores=NC, num_subcores=NS
)


@functools.partial(
    pl.kernel,
    out_type=jax.ShapeDtypeStruct((B, D), jnp.float32),
    mesh=_mesh,
    scratch_types=[
        pltpu.VMEM((RW, SP), jnp.int32),
        pltpu.VMEM((RW, SP), jnp.int32),
        pltpu.VMEM((SP, D), jnp.float32),
        pltpu.VMEM((RW, D), jnp.float32),
        pltpu.SemaphoreType.DMA,
    ],
    compiler_params=pltpu.CompilerParams(use_tc_tiling_on_sc=False),
)
def _sc_pool(ids_hbm, mask_hbm, emb_hbm, sum_hbm, ids_v, mask_v, buf_v, out_v, sem):
    wid = lax.axis_index("s") * NC + lax.axis_index("c")
    base = wid * RW

    pltpu.sync_copy(ids_hbm.at[pl.ds(base, RW)], ids_v)
    pltpu.sync_copy(mask_hbm.at[pl.ds(base, RW)], mask_v)

    def mask_row(r, _):
        def mask_chunk(k, _):
            sl = pl.ds(k * L, L)
            ids_v[r, sl] = ids_v[r, sl] * mask_v[r, sl]
            return 0

        return lax.fori_loop(0, NCH, mask_chunk, 0)

    lax.fori_loop(0, RW, mask_row, 0)

    def row(r, _):
        g0 = pltpu.async_copy(
            emb_hbm.at[ids_v.at[r, pl.ds(0, HALF)]], buf_v.at[pl.ds(0, HALF)], sem
        )
        g1 = pltpu.async_copy(
            emb_hbm.at[ids_v.at[r, pl.ds(HALF, HALF)]], buf_v.at[pl.ds(HALF, HALF)], sem
        )
        g0.wait()
        g1.wait()

        def tok(s, carry):
            a0, a1, a2, a3 = carry
            return (
                a0 + buf_v[s, pl.ds(0, L)],
                a1 + buf_v[s, pl.ds(L, L)],
                a2 + buf_v[s, pl.ds(2 * L, L)],
                a3 + buf_v[s, pl.ds(3 * L, L)],
            )

        z = jnp.zeros((L,), jnp.float32)
        a0, a1, a2, a3 = lax.fori_loop(0, SP, tok, (z, z, z, z))
        out_v[r, pl.ds(0, L)] = a0
        out_v[r, pl.ds(L, L)] = a1
        out_v[r, pl.ds(2 * L, L)] = a2
        out_v[r, pl.ds(3 * L, L)] = a3
        return 0

    lax.fori_loop(0, RW, row, 0)
    pltpu.sync_copy(out_v, sum_hbm.at[pl.ds(base, RW)])


def _head_body(sum_ref, mask_ref, emb0_ref, sm_ref, ss_ref, wt_ref, bias_ref, out_ref):
    cnt = jnp.sum(mask_ref[...].astype(jnp.float32), axis=1, keepdims=True)
    sums = sum_ref[...] - (float(SP) - cnt) * emb0_ref[...]
    sums = jnp.where(cnt > 0.0, sums, 0.0)
    pooled = sums / jnp.maximum(cnt, 1e-9)
    nrm = jnp.sqrt(jnp.sum(pooled * pooled, axis=1, keepdims=True))
    pooled = pooled / jnp.maximum(nrm, 1e-32)
    scaled = (pooled - sm_ref[...]) / ss_ref[...]
    out_ref[...] = (
        jnp.dot(scaled, wt_ref[...], preferred_element_type=jnp.float32) + bias_ref[...]
    )


_head = pl.pallas_call(
    _head_body,
    out_shape=jax.ShapeDtypeStruct((B, C), jnp.float32),
)


def kernel(input_ids, attention_mask, embedding, scaler_mean, scaler_scale, W, b):
    ids = input_ids.astype(jnp.int32)
    mask = attention_mask.astype(jnp.int32)
    ids_p = jnp.pad(ids, ((0, 0), (0, SP - S)))
    mask_p = jnp.pad(mask, ((0, 0), (0, SP - S)))
    sums = _sc_pool(ids_p, mask_p, embedding)
    emb0 = lax.slice(embedding, (0, 0), (1, D))
    return _head(
        sums,
        mask,
        emb0,
        scaler_mean.reshape(1, D),
        scaler_scale.reshape(1, D),
        W.T,
        b.reshape(1, C),
    )


# speedup vs baseline: 1.0001x; 1.0001x over previous
import functools

import jax
import jax.numpy as jnp
from jax import lax
from jax.experimental import pallas as pl
from jax.experimental.pallas import tpu as pltpu
from jax.experimental.pallas import tpu_sc as plsc

B = 4096
S = 200
D = 64
C = 2
L = 16
SP = 208
NC = 2
NS = 16
NW = NC * NS
RW = B // NW
HALF = SP // 2
NCH = SP // L
G = 2
NG = RW // G
UN = 8

_mesh = plsc.VectorSubcoreMesh(
    core_axis_name="c", subcore_axis_name="s", num_cores=NC, num_subcores=NS
)


@functools.partial(
    pl.kernel,
    out_type=jax.ShapeDtypeStruct((B, D), jnp.float32),
    mesh=_mesh,
    scratch_types=[
        pltpu.VMEM((RW, SP), jnp.int32),
        pltpu.VMEM((RW, SP), jnp.int32),
        pltpu.VMEM((2 * G * SP, D), jnp.float32),
        pltpu.VMEM((RW, D), jnp.float32),
        pltpu.SemaphoreType.DMA,
        pltpu.SemaphoreType.DMA,
    ],
    compiler_params=pltpu.CompilerParams(use_tc_tiling_on_sc=False),
)
def _sc_pool(ids_hbm, mask_hbm, emb_hbm, sum_hbm, ids_v, mask_v, buf_v, out_v,
             sem0, sem1):
    wid = lax.axis_index("s") * NC + lax.axis_index("c")
    base = wid * RW
    sems = (sem0, sem1)

    pltpu.sync_copy(ids_hbm.at[pl.ds(base, RW)], ids_v)
    pltpu.sync_copy(mask_hbm.at[pl.ds(base, RW)], mask_v)

    def mask_row(r, _):
        for k in range(NCH):
            sl = pl.ds(k * L, L)
            ids_v[r, sl] = ids_v[r, sl] * mask_v[r, sl]
        return 0

    lax.fori_loop(0, RW, mask_row, 0)

    def gathers(g, nb):
        cps = []
        for j in range(G):
            for h in range(2):
                cps.append(pltpu.make_async_copy(
                    emb_hbm.at[ids_v.at[g * G + j, pl.ds(h * HALF, HALF)]],
                    buf_v.at[pl.ds(nb * G * SP + j * SP + h * HALF, HALF)],
                    sems[nb],
                ))
        return cps

    def issue(g, nb):
        for cp in gathers(g, nb):
            cp.start()

    def drain(g, nb):
        for cp in gathers(g, nb):
            cp.wait()

    def consume(g, nb):
        for j in range(G):
            def tk(t, carry):
                a0, a1, a2, a3 = carry
                for u in range(UN):
                    s = nb * G * SP + j * SP + t * UN + u
                    a0 = a0 + buf_v[s, pl.ds(0, L)]
                    a1 = a1 + buf_v[s, pl.ds(L, L)]
                    a2 = a2 + buf_v[s, pl.ds(2 * L, L)]
                    a3 = a3 + buf_v[s, pl.ds(3 * L, L)]
                return a0, a1, a2, a3

            z = jnp.zeros((L,), jnp.float32)
            a0, a1, a2, a3 = lax.fori_loop(0, SP // UN, tk, (z, z, z, z))
            row = g * G + j
            out_v[row, pl.ds(0, L)] = a0
            out_v[row, pl.ds(L, L)] = a1
            out_v[row, pl.ds(2 * L, L)] = a2
            out_v[row, pl.ds(3 * L, L)] = a3

    issue(0, 0)
    issue(1, 1)

    def body(i, _):
        g0 = 2 * i
        g1 = 2 * i + 1
        drain(g0, 0)
        consume(g0, 0)

        @pl.when(g0 + 2 < NG)
        def _():
            issue(g0 + 2, 0)

        drain(g1, 1)
        consume(g1, 1)

        @pl.when(g1 + 2 < NG)
        def _():
            issue(g1 + 2, 1)

        return 0

    lax.fori_loop(0, NG // 2, body, 0)
    pltpu.sync_copy(out_v, sum_hbm.at[pl.ds(base, RW)])


def _head_body(sum_ref, mask_ref, emb0_ref, sm_ref, ss_ref, wt_ref, bias_ref, out_ref):
    cnt = jnp.sum(mask_ref[...].astype(jnp.float32), axis=1, keepdims=True)
    sums = sum_ref[...] - (float(SP) - cnt) * emb0_ref[...]
    sums = jnp.where(cnt > 0.0, sums, 0.0)
    pooled = sums / jnp.maximum(cnt, 1e-9)
    nrm = jnp.sqrt(jnp.sum(pooled * pooled, axis=1, keepdims=True))
    pooled = pooled / jnp.maximum(nrm, 1e-32)
    scaled = (pooled - sm_ref[...]) / ss_ref[...]
    out_ref[...] = (
        jnp.dot(scaled, wt_ref[...], preferred_element_type=jnp.float32) + bias_ref[...]
    )


_head = pl.pallas_call(
    _head_body,
    out_shape=jax.ShapeDtypeStruct((B, C), jnp.float32),
)


def kernel(input_ids, attention_mask, embedding, scaler_mean, scaler_scale, W, b):
    ids = input_ids.astype(jnp.int32)
    mask = attention_mask.astype(jnp.int32)
    ids_p = jnp.pad(ids, ((0, 0), (0, SP - S)))
    mask_p = jnp.pad(mask, ((0, 0), (0, SP - S)))
    sums = _sc_pool(ids_p, mask_p, embedding)
    emb0 = lax.slice(embedding, (0, 0), (1, D))
    return _head(
        sums,
        mask,
        emb0,
        scaler_mean.reshape(1, D),
        scaler_scale.reshape(1, D),
        W.T,
        b.reshape(1, C),
    )


# speedup vs baseline: 12.5560x; 12.5549x over previous
import functools

import jax
import jax.numpy as jnp
from jax import lax
from jax.experimental import pallas as pl
from jax.experimental.pallas import tpu as pltpu
from jax.experimental.pallas import tpu_sc as plsc

B = 4096
S = 200
D = 64
C = 2
L = 16
SP = 208
NC = 2
NS = 16
NW = NC * NS
RW = B // NW
NCH = SP // L
UN = 8

_mesh = plsc.VectorSubcoreMesh(
    core_axis_name="c", subcore_axis_name="s", num_cores=NC, num_subcores=NS
)


@functools.partial(
    pl.kernel,
    out_type=jax.ShapeDtypeStruct((B, D), jnp.float32),
    mesh=_mesh,
    scratch_types=[
        pltpu.VMEM((RW, SP), jnp.int32),
        pltpu.VMEM((RW, SP), jnp.int32),
        pltpu.VMEM((2 * SP, D), jnp.float32),
        pltpu.VMEM((RW, D), jnp.float32),
        pltpu.SemaphoreType.DMA,
        pltpu.SemaphoreType.DMA,
    ],
    compiler_params=pltpu.CompilerParams(
        use_tc_tiling_on_sc=False, needs_layout_passes=False
    ),
)
def _sc_pool(ids_hbm, mask_hbm, emb_hbm, sum_hbm, ids_v, mask_v, buf_v, out_v,
             sem0, sem1):
    wid = lax.axis_index("s") * NC + lax.axis_index("c")
    base = wid * RW
    sems = (sem0, sem1)

    pltpu.sync_copy(ids_hbm.at[pl.ds(base, RW)], ids_v)
    pltpu.sync_copy(mask_hbm.at[pl.ds(base, RW)], mask_v)

    def compact(r):
        rv = jnp.broadcast_to(r.astype(jnp.int32), (L,))
        off = jnp.int32(0)
        for k in range(NCH):
            sl = pl.ds(k * L, L)
            idc = ids_v[r, sl]
            mc = mask_v[r, sl]
            pos = plsc.cumsum(mc) - mc + off
            plsc.store_scatter(ids_v, [rv, pos], idc, mask=mc != 0)
            off = off + jnp.sum(mc)
        return off

    def chunk_copy(r, slot, c):
        co = pl.multiple_of(c * L, L)
        return pltpu.make_async_copy(
            emb_hbm.at[ids_v.at[r, pl.ds(co, L)]],
            buf_v.at[pl.ds(slot * SP, SP)].at[pl.ds(co, L)],
            sems[slot],
        )

    def issue(r, slot, m):
        nch = (m + L - 1) // L

        def ic(c, _):
            chunk_copy(r, slot, c).start()
            return 0

        lax.fori_loop(0, nch, ic, 0)

    def drain(r, slot, m):
        nch = (m + L - 1) // L

        def dc(c, _):
            chunk_copy(r, slot, c).wait()
            return 0

        lax.fori_loop(0, nch, dc, 0)

    def accum(r, slot, m):
        n8 = m // UN

        def t8(t, carry):
            a0, a1, a2, a3 = carry
            for u in range(UN):
                s = slot * SP + t * UN + u
                a0 = a0 + buf_v[s, pl.ds(0, L)]
                a1 = a1 + buf_v[s, pl.ds(L, L)]
                a2 = a2 + buf_v[s, pl.ds(2 * L, L)]
                a3 = a3 + buf_v[s, pl.ds(3 * L, L)]
            return a0, a1, a2, a3

        z = jnp.zeros((L,), jnp.float32)
        acc = lax.fori_loop(0, n8, t8, (z, z, z, z))

        def t1(s_, carry):
            a0, a1, a2, a3 = carry
            s = slot * SP + s_
            return (
                a0 + buf_v[s, pl.ds(0, L)],
                a1 + buf_v[s, pl.ds(L, L)],
                a2 + buf_v[s, pl.ds(2 * L, L)],
                a3 + buf_v[s, pl.ds(3 * L, L)],
            )

        a0, a1, a2, a3 = lax.fori_loop(n8 * UN, m, t1, acc)
        out_v[r, pl.ds(0, L)] = a0
        out_v[r, pl.ds(L, L)] = a1
        out_v[r, pl.ds(2 * L, L)] = a2
        out_v[r, pl.ds(3 * L, L)] = a3

    def prep(rnext, slot):
        safe = jnp.where(rnext < RW, rnext, 0)
        m = compact(safe)

        @pl.when(rnext < RW)
        def _():
            issue(rnext, slot, m)

        return m

    m0 = compact(jnp.int32(0))
    issue(jnp.int32(0), 0, m0)
    m1 = compact(jnp.int32(1))
    issue(jnp.int32(1), 1, m1)

    def body(i, carry):
        ma, mb = carry
        ra = 2 * i
        rb = 2 * i + 1
        drain(ra, 0, ma)
        accum(ra, 0, ma)
        mc = prep(ra + 2, 0)
        drain(rb, 1, mb)
        accum(rb, 1, mb)
        md = prep(rb + 2, 1)
        return mc, md

    lax.fori_loop(0, RW // 2, body, (m0, m1))
    pltpu.sync_copy(out_v, sum_hbm.at[pl.ds(base, RW)])


def _head_body(sum_ref, mask_ref, sm_ref, ss_ref, wt_ref, bias_ref, out_ref):
    cnt = jnp.sum(mask_ref[...].astype(jnp.float32), axis=1, keepdims=True)
    sums = sum_ref[...]
    pooled = sums / jnp.maximum(cnt, 1e-9)
    nrm = jnp.sqrt(jnp.sum(pooled * pooled, axis=1, keepdims=True))
    pooled = pooled / jnp.maximum(nrm, 1e-32)
    scaled = (pooled - sm_ref[...]) / ss_ref[...]
    out_ref[...] = (
        jnp.dot(scaled, wt_ref[...], preferred_element_type=jnp.float32) + bias_ref[...]
    )


_head = pl.pallas_call(
    _head_body,
    out_shape=jax.ShapeDtypeStruct((B, C), jnp.float32),
)


def kernel(input_ids, attention_mask, embedding, scaler_mean, scaler_scale, W, b):
    ids = input_ids.astype(jnp.int32)
    mask = attention_mask.astype(jnp.int32)
    ids_p = jnp.pad(ids, ((0, 0), (0, SP - S)))
    mask_p = jnp.pad(mask, ((0, 0), (0, SP - S)))
    sums = _sc_pool(ids_p, mask_p, embedding)
    return _head(
        sums,
        mask,
        scaler_mean.reshape(1, D),
        scaler_scale.reshape(1, D),
        W.T,
        b.reshape(1, C),
    )


# speedup vs baseline: 12.9488x; 1.0313x over previous
import functools

import jax
import jax.numpy as jnp
from jax import lax
from jax.experimental import pallas as pl
from jax.experimental.pallas import tpu as pltpu
from jax.experimental.pallas import tpu_sc as plsc

B = 4096
S = 200
D = 64
C = 2
L = 16
SP = 208
NC = 2
NS = 16
NW = NC * NS
RW = B // NW
NCH = SP // L
UN = 8

_mesh = plsc.VectorSubcoreMesh(
    core_axis_name="c", subcore_axis_name="s", num_cores=NC, num_subcores=NS
)


@functools.partial(
    pl.kernel,
    out_type=jax.ShapeDtypeStruct((B, D), jnp.float32),
    mesh=_mesh,
    scratch_types=[
        pltpu.VMEM((RW, SP), jnp.int32),
        pltpu.VMEM((RW, SP), jnp.int32),
        pltpu.VMEM((4 * SP, D), jnp.float32),
        pltpu.VMEM((RW, D), jnp.float32),
        pltpu.SemaphoreType.DMA,
        pltpu.SemaphoreType.DMA,
        pltpu.SemaphoreType.DMA,
        pltpu.SemaphoreType.DMA,
    ],
    compiler_params=pltpu.CompilerParams(
        use_tc_tiling_on_sc=False, needs_layout_passes=False
    ),
)
def _sc_pool(ids_hbm, mask_hbm, emb_hbm, sum_hbm, ids_v, mask_v, buf_v, out_v,
             sem0, sem1, sem2, sem3):
    wid = lax.axis_index("s") * NC + lax.axis_index("c")
    base = wid * RW
    sems = (sem0, sem1, sem2, sem3)

    pltpu.sync_copy(ids_hbm.at[pl.ds(base, RW)], ids_v)
    pltpu.sync_copy(mask_hbm.at[pl.ds(base, RW)], mask_v)

    def compact(r):
        rv = jnp.broadcast_to(r.astype(jnp.int32), (L,))
        off = jnp.int32(0)
        for k in range(NCH):
            sl = pl.ds(k * L, L)
            idc = ids_v[r, sl]
            mc = mask_v[r, sl]
            pos = plsc.cumsum(mc) - mc + off
            plsc.store_scatter(ids_v, [rv, pos], idc, mask=mc != 0)
            off = off + jnp.sum(mc)
        return off

    def chunk_copy(r, slot, c):
        co = pl.multiple_of(c * L, L)
        return pltpu.make_async_copy(
            emb_hbm.at[ids_v.at[r, pl.ds(co, L)]],
            buf_v.at[pl.ds(slot * SP, SP)].at[pl.ds(co, L)],
            sems[slot],
        )

    def issue(r, slot, m):
        nch = (m + L - 1) // L

        def ic(c, _):
            chunk_copy(r, slot, c).start()
            return 0

        lax.fori_loop(0, nch, ic, 0)

    def drain(r, slot, m):
        nch = (m + L - 1) // L

        def dc(c, _):
            chunk_copy(r, slot, c).wait()
            return 0

        lax.fori_loop(0, nch, dc, 0)

    def accum(r, slot, m):
        n8 = m // UN

        def t8(t, carry):
            a0, a1, a2, a3 = carry
            for u in range(UN):
                s = slot * SP + t * UN + u
                a0 = a0 + buf_v[s, pl.ds(0, L)]
                a1 = a1 + buf_v[s, pl.ds(L, L)]
                a2 = a2 + buf_v[s, pl.ds(2 * L, L)]
                a3 = a3 + buf_v[s, pl.ds(3 * L, L)]
            return a0, a1, a2, a3

        z = jnp.zeros((L,), jnp.float32)
        acc = lax.fori_loop(0, n8, t8, (z, z, z, z))

        def t1(s_, carry):
            a0, a1, a2, a3 = carry
            s = slot * SP + s_
            return (
                a0 + buf_v[s, pl.ds(0, L)],
                a1 + buf_v[s, pl.ds(L, L)],
                a2 + buf_v[s, pl.ds(2 * L, L)],
                a3 + buf_v[s, pl.ds(3 * L, L)],
            )

        a0, a1, a2, a3 = lax.fori_loop(n8 * UN, m, t1, acc)
        out_v[r, pl.ds(0, L)] = a0
        out_v[r, pl.ds(L, L)] = a1
        out_v[r, pl.ds(2 * L, L)] = a2
        out_v[r, pl.ds(3 * L, L)] = a3

    def prep(rnext, slot):
        safe = jnp.where(rnext < RW, rnext, 0)
        m = compact(safe)

        @pl.when(rnext < RW)
        def _():
            issue(rnext, slot, m)

        return m

    NSLOT = 4
    ms = []
    for j in range(NSLOT):
        mj = compact(jnp.int32(j))
        issue(jnp.int32(j), j, mj)
        ms.append(mj)

    def body(i, carry):
        carry = list(carry)
        for j in range(NSLOT):
            r = NSLOT * i + j
            drain(r, j, carry[j])
            accum(r, j, carry[j])
            carry[j] = prep(r + NSLOT, j)
        return tuple(carry)

    lax.fori_loop(0, RW // NSLOT, body, tuple(ms))
    pltpu.sync_copy(out_v, sum_hbm.at[pl.ds(base, RW)])


def _head_body(sum_ref, mask_ref, sm_ref, ss_ref, wt_ref, bias_ref, out_ref):
    cnt = jnp.sum(mask_ref[...].astype(jnp.float32), axis=1, keepdims=True)
    sums = sum_ref[...]
    pooled = sums / jnp.maximum(cnt, 1e-9)
    nrm = jnp.sqrt(jnp.sum(pooled * pooled, axis=1, keepdims=True))
    pooled = pooled / jnp.maximum(nrm, 1e-32)
    scaled = (pooled - sm_ref[...]) / ss_ref[...]
    out_ref[...] = (
        jnp.dot(scaled, wt_ref[...], preferred_element_type=jnp.float32) + bias_ref[...]
    )


_head = pl.pallas_call(
    _head_body,
    out_shape=jax.ShapeDtypeStruct((B, C), jnp.float32),
)


def kernel(input_ids, attention_mask, embedding, scaler_mean, scaler_scale, W, b):
    ids = input_ids.astype(jnp.int32)
    mask = attention_mask.astype(jnp.int32)
    ids_p = jnp.pad(ids, ((0, 0), (0, SP - S)))
    mask_p = jnp.pad(mask, ((0, 0), (0, SP - S)))
    sums = _sc_pool(ids_p, mask_p, embedding)
    return _head(
        sums,
        mask,
        scaler_mean.reshape(1, D),
        scaler_scale.reshape(1, D),
        W.T,
        b.reshape(1, C),
    )


# speedup vs baseline: 14.2699x; 1.1020x over previous
import functools

import jax
import jax.numpy as jnp
from jax import lax
from jax.experimental import pallas as pl
from jax.experimental.pallas import tpu as pltpu
from jax.experimental.pallas import tpu_sc as plsc

B = 4096
VOCAB = 1000000
S = 200
D = 64
C = 2
L = 16
SP = 208
NC = 2
NS = 16
NW = NC * NS
RW = B // NW
NCH = SP // L
UN = 8

_mesh = plsc.VectorSubcoreMesh(
    core_axis_name="c", subcore_axis_name="s", num_cores=NC, num_subcores=NS
)


@functools.partial(
    pl.kernel,
    out_type=jax.ShapeDtypeStruct((B, D), jnp.float32),
    mesh=_mesh,
    scratch_types=[
        pltpu.VMEM((RW, SP), jnp.int32),
        pltpu.VMEM((RW, SP), jnp.int32),
        pltpu.VMEM((4 * SP, D), jnp.float32),
        pltpu.VMEM((RW, D), jnp.float32),
        pltpu.SemaphoreType.DMA,
        pltpu.SemaphoreType.DMA,
        pltpu.SemaphoreType.DMA,
        pltpu.SemaphoreType.DMA,
    ],
    compiler_params=pltpu.CompilerParams(
        use_tc_tiling_on_sc=False, needs_layout_passes=False
    ),
)
def _sc_pool(ids_hbm, mask_hbm, emb_hbm, sum_hbm, ids_v, mask_v, buf_v, out_v,
             sem0, sem1, sem2, sem3):
    wid = lax.axis_index("s") * NC + lax.axis_index("c")
    base = wid * RW
    sems = (sem0, sem1, sem2, sem3)

    pltpu.sync_copy(ids_hbm.at[pl.ds(base, RW)], ids_v)
    pltpu.sync_copy(mask_hbm.at[pl.ds(base, RW)], mask_v)

    def compact(r):
        rv = jnp.broadcast_to(r.astype(jnp.int32), (L,))
        off = jnp.int32(0)
        for k in range(NCH):
            sl = pl.ds(k * L, L)
            idc = ids_v[r, sl]
            mc = mask_v[r, sl]
            pos = plsc.cumsum(mc) - mc + off
            plsc.store_scatter(ids_v, [rv, pos], idc + idc, mask=mc != 0)
            off = off + jnp.sum(mc)
        return off

    def chunk_copy(r, slot, c):
        co = pl.multiple_of(c * L, L)
        return pltpu.make_async_copy(
            emb_hbm.at[ids_v.at[r, pl.ds(co, L)]],
            buf_v.at[pl.ds(slot * SP, SP)].at[pl.ds(co, L)],
            sems[slot],
        )

    def issue(r, slot, m):
        nch = (m + L - 1) // L

        def ic(c, _):
            chunk_copy(r, slot, c).start()
            return 0

        lax.fori_loop(0, nch, ic, 0)

    def drain(r, slot, m):
        nch = (m + L - 1) // L

        def dc(c, _):
            chunk_copy(r, slot, c).wait()
            return 0

        lax.fori_loop(0, nch, dc, 0)

    def accum(r, slot, m):
        n8 = m // UN

        def t8(t, carry):
            a0, a1, a2, a3 = carry
            for u in range(UN):
                s = slot * SP + t * UN + u
                a0 = a0 + buf_v[s, pl.ds(0, L)]
                a1 = a1 + buf_v[s, pl.ds(L, L)]
                a2 = a2 + buf_v[s, pl.ds(2 * L, L)]
                a3 = a3 + buf_v[s, pl.ds(3 * L, L)]
            return a0, a1, a2, a3

        z = jnp.zeros((L,), jnp.float32)
        acc = lax.fori_loop(0, n8, t8, (z, z, z, z))

        def t1(s_, carry):
            a0, a1, a2, a3 = carry
            s = slot * SP + s_
            return (
                a0 + buf_v[s, pl.ds(0, L)],
                a1 + buf_v[s, pl.ds(L, L)],
                a2 + buf_v[s, pl.ds(2 * L, L)],
                a3 + buf_v[s, pl.ds(3 * L, L)],
            )

        a0, a1, a2, a3 = lax.fori_loop(n8 * UN, m, t1, acc)
        out_v[r, pl.ds(0, L)] = a0
        out_v[r, pl.ds(L, L)] = a1
        out_v[r, pl.ds(2 * L, L)] = a2
        out_v[r, pl.ds(3 * L, L)] = a3

    def prep(rnext, slot):
        safe = jnp.where(rnext < RW, rnext, 0)
        m = compact(safe)

        @pl.when(rnext < RW)
        def _():
            issue(rnext, slot, m)

        return m

    NSLOT = 4
    ms = []
    for j in range(NSLOT):
        mj = compact(jnp.int32(j))
        issue(jnp.int32(j), j, mj)
        ms.append(mj)

    def body(i, carry):
        carry = list(carry)
        for j in range(NSLOT):
            r = NSLOT * i + j
            drain(r, j, carry[j])
            accum(r, j, carry[j])
            carry[j] = prep(r + NSLOT, j)
        return tuple(carry)

    lax.fori_loop(0, RW // NSLOT, body, tuple(ms))
    pltpu.sync_copy(out_v, sum_hbm.at[pl.ds(base, RW)])


def _head_body(sum_ref, mask_ref, sm_ref, ss_ref, wt_ref, bias_ref, out_ref):
    cnt = jnp.sum(mask_ref[...].astype(jnp.float32), axis=1, keepdims=True)
    sums = sum_ref[...]
    pooled = sums / jnp.maximum(cnt, 1e-9)
    nrm = jnp.sqrt(jnp.sum(pooled * pooled, axis=1, keepdims=True))
    pooled = pooled / jnp.maximum(nrm, 1e-32)
    scaled = (pooled - sm_ref[...]) / ss_ref[...]
    out_ref[...] = (
        jnp.dot(scaled, wt_ref[...], preferred_element_type=jnp.float32) + bias_ref[...]
    )


_head = pl.pallas_call(
    _head_body,
    out_shape=jax.ShapeDtypeStruct((B, C), jnp.float32),
)


def kernel(input_ids, attention_mask, embedding, scaler_mean, scaler_scale, W, b):
    ids = input_ids.astype(jnp.int32)
    mask = attention_mask.astype(jnp.int32)
    ids_p = jnp.pad(ids, ((0, 0), (0, SP - S)))
    mask_p = jnp.pad(mask, ((0, 0), (0, SP - S)))
    emb2 = jnp.pad(embedding, ((0, 0), (0, D))).reshape(2 * VOCAB, D)
    sums = _sc_pool(ids_p, mask_p, emb2)
    return _head(
        sums,
        mask,
        scaler_mean.reshape(1, D),
        scaler_scale.reshape(1, D),
        W.T,
        b.reshape(1, C),
    )
